# Initial kernel scaffold; baseline (speedup 1.0000x reference)
import jax
import jax.numpy as jnp
from jax.experimental import pallas as pl


def kernel(x, indices, size):
    raise NotImplementedError("write your pallas kernel here")


# speedup vs baseline: 3.8272x; 3.8272x over previous
import functools

import jax
import jax.numpy as jnp
from jax import lax
from jax.experimental import pallas as pl
from jax.experimental.pallas import tpu as pltpu
from jax.experimental.pallas import tpu_sc as plsc

L = 16
B, C, H, W = 4, 96, 192, 192
HW = H * W
NSLICE = B * C
N = NSLICE * HW
UP = 385
UPHW = UP * UP
OUTW = 384
OUT_HW = OUTW * OUTW
OUT_N = NSLICE * OUT_HW

NWORKERS = 32
SLICES_PER_W = NSLICE // NWORKERS
ROWS_PER_CHUNK = 64
NCH = OUTW // ROWS_PER_CHUNK
CHUNK = ROWS_PER_CHUNK * OUTW
NCHUNKS = NSLICE * NCH

EBUF = 16384
PAD = EBUF + 8

MAGIC385 = 21791
SHIFT385 = 23


def _div385(p):
    pu = p.astype(jnp.uint32)
    q = ((pu * jnp.uint32(MAGIC385)) >> jnp.uint32(SHIFT385)).astype(jnp.int32)
    r = p - q * UP
    neg = r < 0
    q = jnp.where(neg, q - 1, q)
    r = jnp.where(neg, r + UP, r)
    return q, r


def _sget(vref, i):
    vec = vref[pl.ds((i >> 4) * L, L)]
    lane = lax.iota(jnp.int32, L)
    sel = jnp.where(lane == (i & (L - 1)), vec, jnp.int32(-2147483648))
    return jnp.max(sel)


def _consume_body(sk_hbm, sv_hbm, starts_hbm, out_hbm,
                  kbuf, vbuf, stg0, stg1, vrow, in_sem, sem0, sem1):
    cid = lax.axis_index("c")
    sid = lax.axis_index("s")
    wid = sid * 2 + cid

    pltpu.sync_copy(starts_hbm.at[wid], vrow)

    def slice_body(si, carry):
        s_chunk = wid * SLICES_PER_W + si
        kbase = s_chunk * UPHW
        dbase_slice = s_chunk * OUT_HW

        for ch in range(NCH):
            stg = stg0 if ch % 2 == 0 else stg1
            sem = sem0 if ch % 2 == 0 else sem1
            row0 = ch * ROWS_PER_CHUNK

            a = _sget(vrow, si * NCH + ch)
            b = _sget(vrow, si * NCH + ch + 1)
            al = pl.multiple_of(a & ~7, 8)
            n_eff = jnp.minimum(b - a, EBUF - 8)
            nvec = (n_eff + (L - 1)) // L

            cpk = pltpu.async_copy(sk_hbm.at[pl.ds(al, EBUF)],
                                   kbuf.at[pl.ds(0, EBUF)], in_sem)
            cpv = pltpu.async_copy(sv_hbm.at[pl.ds(al, EBUF)],
                                   vbuf.at[pl.ds(0, EBUF)], in_sem)

            if ch >= 2:
                pltpu.make_async_copy(
                    out_hbm.at[pl.ds(0, CHUNK)], stg, sem).wait()
            else:
                @pl.when(si > 0)
                def _drain():
                    pltpu.make_async_copy(
                        out_hbm.at[pl.ds(0, CHUNK)], stg, sem).wait()

            def zbody(j, c2):
                stg[pl.ds(j * L, L)] = jnp.zeros((L,), jnp.float32)
                return c2

            lax.fori_loop(0, CHUNK // L, zbody, None)

            cpk.wait()
            cpv.wait()

            shift = a - al
            p2base = kbase + row0 * UP

            def vbody(j, c2):
                off = shift + j * L
                k = kbuf[pl.ds(off, L)]
                kn = kbuf[pl.ds(off + 1, L)]
                v = vbuf[pl.ds(off, L)]
                lane = jax.lax.iota(jnp.int32, L) + (j * L)
                win = (k != kn) & (lane < n_eff)
                p2 = k - p2base
                r2, cc = _div385(p2)
                valid = win & (r2 < ROWS_PER_CHUNK) & (cc < OUTW)
                ldest = r2 * OUTW + cc
                plsc.store_scatter(stg, [ldest], v, mask=valid)
                return c2

            lax.fori_loop(0, nvec, vbody, None)

            dbase = pl.multiple_of(dbase_slice + row0 * OUTW, 8)
            pltpu.async_copy(stg, out_hbm.at[pl.ds(dbase, CHUNK)], sem)
        return carry

    lax.fori_loop(0, SLICES_PER_W, slice_body, None)

    pltpu.make_async_copy(out_hbm.at[pl.ds(0, CHUNK)], stg0, sem0).wait()
    pltpu.make_async_copy(out_hbm.at[pl.ds(0, CHUNK)], stg1, sem1).wait()


def kernel(x, indices, size):
    del size
    idx = indices.reshape(NSLICE, HW)
    bc = (jnp.arange(NSLICE, dtype=jnp.int32) * UPHW)[:, None]
    keys = (idx + bc).reshape(-1)
    vals = x.reshape(-1)
    sk, sv = lax.sort_key_val(keys, vals, is_stable=False)
    sk_pad = jnp.concatenate(
        [sk, jnp.full((PAD,), jnp.int32(2**31 - 1))])
    sv_pad = jnp.concatenate([sv, jnp.zeros((PAD,), jnp.float32)])

    g = jnp.arange(NCHUNKS + 1, dtype=jnp.int32)
    slice_of = g // NCH
    row0_of = (g % NCH) * ROWS_PER_CHUNK
    bkeys = slice_of * UPHW + row0_of * UP
    starts = jnp.searchsorted(sk, bkeys, side="left").astype(jnp.int32)
    per_w = NCHUNKS // NWORKERS
    rows = starts[(jnp.arange(NWORKERS)[:, None] * per_w
                   + jnp.arange(per_w + 1)[None, :]).reshape(-1)]
    starts_arr = jnp.concatenate(
        [rows.reshape(NWORKERS, per_w + 1),
         jnp.zeros((NWORKERS, 80 - (per_w + 1)), jnp.int32)], axis=1)

    mesh = plsc.VectorSubcoreMesh(core_axis_name="c", subcore_axis_name="s")
    out = pl.kernel(
        _consume_body,
        out_type=jax.ShapeDtypeStruct((OUT_N,), jnp.float32),
        mesh=mesh,
        compiler_params=pltpu.CompilerParams(needs_layout_passes=False),
        scratch_types=[
            pltpu.VMEM((EBUF + L,), jnp.int32),
            pltpu.VMEM((EBUF + L,), jnp.float32),
            pltpu.VMEM((CHUNK,), jnp.float32),
            pltpu.VMEM((CHUNK,), jnp.float32),
            pltpu.VMEM((80,), jnp.int32),
            pltpu.SemaphoreType.DMA,
            pltpu.SemaphoreType.DMA,
            pltpu.SemaphoreType.DMA,
        ],
    )(sk_pad, sv_pad, starts_arr)
    return out.reshape(B, C, OUTW, OUTW)


# speedup vs baseline: 3.8415x; 1.0037x over previous
import functools

import jax
import jax.numpy as jnp
from jax import lax
from jax.experimental import pallas as pl
from jax.experimental.pallas import tpu as pltpu
from jax.experimental.pallas import tpu_sc as plsc

L = 16
B, C, H, W = 4, 96, 192, 192
HW = H * W
NSLICE = B * C
N = NSLICE * HW
UP = 385
UPHW = UP * UP
OUTW = 384
OUT_HW = OUTW * OUTW
OUT_N = NSLICE * OUT_HW

NWORKERS = 32
SLICES_PER_W = NSLICE // NWORKERS
ROWS_PER_CHUNK = 64
NCH = OUTW // ROWS_PER_CHUNK
CHUNK = ROWS_PER_CHUNK * OUTW
NCHUNKS = NSLICE * NCH

EBUF = 8192

MAGIC385 = 21791
SHIFT385 = 23


def _div385(p):
    pu = p.astype(jnp.uint32)
    q = ((pu * jnp.uint32(MAGIC385)) >> jnp.uint32(SHIFT385)).astype(jnp.int32)
    r = p - q * UP
    neg = r < 0
    q = jnp.where(neg, q - 1, q)
    r = jnp.where(neg, r + UP, r)
    return q, r


def _sget(vref, i):
    vec = vref[pl.ds((i >> 4) * L, L)]
    lane = lax.iota(jnp.int32, L)
    sel = jnp.where(lane == (i & (L - 1)), vec, jnp.int32(-2147483648))
    return jnp.max(sel)


def _consume_body(sk_hbm, sv_hbm, starts_hbm, out_hbm,
                  kbuf, vbuf, stg0, stg1, vrow, in_sem, sem0, sem1):
    cid = lax.axis_index("c")
    sid = lax.axis_index("s")
    wid = sid * 2 + cid

    pltpu.sync_copy(starts_hbm.at[wid], vrow)

    kbuf[pl.ds(EBUF, L)] = jnp.full((L,), jnp.int32(2**31 - 1))

    def slice_body(si, carry):
        s_chunk = wid * SLICES_PER_W + si
        kbase = s_chunk * UPHW
        dbase_slice = s_chunk * OUT_HW

        for ch in range(NCH):
            stg = stg0 if ch % 2 == 0 else stg1
            sem = sem0 if ch % 2 == 0 else sem1
            row0 = ch * ROWS_PER_CHUNK

            a = _sget(vrow, si * NCH + ch)
            b = _sget(vrow, si * NCH + ch + 1)
            al = pl.multiple_of(jnp.minimum(a & ~7, N - EBUF), 8)
            n_eff = jnp.minimum(b - a, EBUF - 8)
            nvec = (n_eff + (L - 1)) // L

            cpk = pltpu.async_copy(sk_hbm.at[pl.ds(al, EBUF)],
                                   kbuf.at[pl.ds(0, EBUF)], in_sem)
            cpv = pltpu.async_copy(sv_hbm.at[pl.ds(al, EBUF)],
                                   vbuf.at[pl.ds(0, EBUF)], in_sem)

            if ch >= 2:
                pltpu.make_async_copy(
                    out_hbm.at[pl.ds(0, CHUNK)], stg, sem).wait()
            else:
                @pl.when(si > 0)
                def _drain():
                    pltpu.make_async_copy(
                        out_hbm.at[pl.ds(0, CHUNK)], stg, sem).wait()

            def zbody(j, c2):
                stg[pl.ds(j * L, L)] = jnp.zeros((L,), jnp.float32)
                return c2

            lax.fori_loop(0, CHUNK // L, zbody, None)

            cpk.wait()
            cpv.wait()

            shift = a - al
            p2base = kbase + row0 * UP

            def vbody(j, c2):
                off = shift + j * L
                k = kbuf[pl.ds(off, L)]
                kn = kbuf[pl.ds(off + 1, L)]
                v = vbuf[pl.ds(off, L)]
                lane = jax.lax.iota(jnp.int32, L) + (j * L)
                win = (k != kn) & (lane < n_eff)
                p2 = k - p2base
                r2, cc = _div385(p2)
                valid = win & (r2 < ROWS_PER_CHUNK) & (cc < OUTW)
                ldest = r2 * OUTW + cc
                plsc.store_scatter(stg, [ldest], v, mask=valid)
                return c2

            lax.fori_loop(0, nvec, vbody, None)

            dbase = pl.multiple_of(dbase_slice + row0 * OUTW, 8)
            pltpu.async_copy(stg, out_hbm.at[pl.ds(dbase, CHUNK)], sem)
        return carry

    lax.fori_loop(0, SLICES_PER_W, slice_body, None)

    pltpu.make_async_copy(out_hbm.at[pl.ds(0, CHUNK)], stg0, sem0).wait()
    pltpu.make_async_copy(out_hbm.at[pl.ds(0, CHUNK)], stg1, sem1).wait()


def kernel(x, indices, size):
    del size
    idx = indices.reshape(NSLICE, HW)
    bc = (jnp.arange(NSLICE, dtype=jnp.int32) * UPHW)[:, None]
    keys = (idx + bc).reshape(-1)
    vals = x.reshape(-1)
    sk, sv = lax.sort_key_val(keys, vals, is_stable=False)

    g = jnp.arange(NCHUNKS + 1, dtype=jnp.int32)
    slice_of = g // NCH
    row0_of = (g % NCH) * ROWS_PER_CHUNK
    bkeys = slice_of * UPHW + row0_of * UP
    starts = jnp.searchsorted(sk, bkeys, side="left").astype(jnp.int32)
    per_w = NCHUNKS // NWORKERS
    rows = starts[(jnp.arange(NWORKERS)[:, None] * per_w
                   + jnp.arange(per_w + 1)[None, :]).reshape(-1)]
    starts_arr = jnp.concatenate(
        [rows.reshape(NWORKERS, per_w + 1),
         jnp.zeros((NWORKERS, 80 - (per_w + 1)), jnp.int32)], axis=1)

    mesh = plsc.VectorSubcoreMesh(core_axis_name="c", subcore_axis_name="s")
    out = pl.kernel(
        _consume_body,
        out_type=jax.ShapeDtypeStruct((OUT_N,), jnp.float32),
        mesh=mesh,
        compiler_params=pltpu.CompilerParams(needs_layout_passes=False),
        scratch_types=[
            pltpu.VMEM((EBUF + L,), jnp.int32),
            pltpu.VMEM((EBUF + L,), jnp.float32),
            pltpu.VMEM((CHUNK,), jnp.float32),
            pltpu.VMEM((CHUNK,), jnp.float32),
            pltpu.VMEM((80,), jnp.int32),
            pltpu.SemaphoreType.DMA,
            pltpu.SemaphoreType.DMA,
            pltpu.SemaphoreType.DMA,
        ],
    )(sk, sv, starts_arr)
    return out.reshape(B, C, OUTW, OUTW)
